# initial kernel scaffold (unmeasured)
import jax
import jax.numpy as jnp
from jax import lax
from jax.experimental import pallas as pl
from jax.experimental.pallas import tpu as pltpu

B, HS, WS, C = 2, 64, 64, 64
GH, GW = 128, 128
EPS = 1e-5

_CompilerParams = getattr(pltpu, "CompilerParams", None) or getattr(
    pltpu, "TPUCompilerParams"
)


def kernel(x, k, Wp):
    def body(x_ref, k_ref, w_ref, out_ref, stats_s, stats_r, pad,
             send_sems, recv_sems):
        px = lax.axis_index("x")
        py = lax.axis_index("y")
        xn = (1 - px, py)
        yn = (px, 1 - py)

        bar = pltpu.get_barrier_semaphore()
        pl.semaphore_signal(bar, inc=1, device_id=xn,
                            device_id_type=pl.DeviceIdType.MESH)
        pl.semaphore_signal(bar, inc=1, device_id=yn,
                            device_id_type=pl.DeviceIdType.MESH)
        pl.semaphore_wait(bar, 2)

        xv = x_ref[...]

        stats_s[0, :, :] = jnp.sum(xv, axis=(1, 2))
        stats_s[1, :, :] = jnp.sum(xv * xv, axis=(1, 2))

        st1 = pltpu.make_async_remote_copy(
            src_ref=stats_s,
            dst_ref=stats_r.at[0],
            send_sem=send_sems.at[0],
            recv_sem=recv_sems.at[0],
            device_id=xn,
            device_id_type=pl.DeviceIdType.MESH,
        )
        st1.start()

        pad[:, 1:HS + 1, 1:WS + 1, :] = xv

        src_r = jnp.where(px == 0, HS, 1)
        dst_r = jnp.where(px == 0, 0, HS + 1)
        row_rdma = pltpu.make_async_remote_copy(
            src_ref=pad.at[:, pl.ds(src_r, 1), pl.ds(1, WS), :],
            dst_ref=pad.at[:, pl.ds(dst_r, 1), pl.ds(1, WS), :],
            send_sem=send_sems.at[2],
            recv_sem=recv_sems.at[2],
            device_id=xn,
            device_id_type=pl.DeviceIdType.MESH,
        )
        row_rdma.start()

        @pl.when(px == 0)
        def _():
            pad[:, 0:1, 1:WS + 1, :] = xv[:, 0:1, :, :]

        @pl.when(px == 1)
        def _():
            pad[:, HS + 1:HS + 2, 1:WS + 1, :] = xv[:, HS - 1:HS, :, :]

        st1.wait()
        stats_s[...] = stats_s[...] + stats_r[0]
        st2 = pltpu.make_async_remote_copy(
            src_ref=stats_s,
            dst_ref=stats_r.at[1],
            send_sem=send_sems.at[1],
            recv_sem=recv_sems.at[1],
            device_id=yn,
            device_id_type=pl.DeviceIdType.MESH,
        )
        st2.start()

        row_rdma.wait()
        src_c = jnp.where(py == 0, WS, 1)
        dst_c = jnp.where(py == 0, 0, WS + 1)
        col_rdma = pltpu.make_async_remote_copy(
            src_ref=pad.at[:, :, pl.ds(src_c, 1), :],
            dst_ref=pad.at[:, :, pl.ds(dst_c, 1), :],
            send_sem=send_sems.at[3],
            recv_sem=recv_sems.at[3],
            device_id=yn,
            device_id_type=pl.DeviceIdType.MESH,
        )
        col_rdma.start()

        @pl.when(py == 0)
        def _():
            pad[:, :, 0:1, :] = pad[:, :, 1:2, :]

        @pl.when(py == 1)
        def _():
            pad[:, :, WS + 1:WS + 2, :] = pad[:, :, WS:WS + 1, :]

        st2.wait()
        total = stats_s[...] + stats_r[1]
        n = float(GH * GW)
        mean = total[0] / n
        var = total[1] / n - mean * mean
        inv = lax.rsqrt(var + EPS)
        mean_b = mean.reshape(B, 1, 1, C)
        inv_b = inv.reshape(B, 1, 1, C)

        col_rdma.wait()
        hp = (pad[...] - mean_b) * inv_b

        conv = jnp.zeros((B, HS, WS, C), jnp.float32)
        for di in range(3):
            for dj in range(3):
                conv = conv + hp[:, di:di + HS, dj:dj + WS, :] * (
                    k_ref[di, dj, :].reshape(1, 1, 1, C)
                )
        a = conv / (1.0 + jnp.exp(-conv))
        proj = jnp.dot(
            a.reshape(B * HS * WS, C), w_ref[...],
            preferred_element_type=jnp.float32,
        )
        out_ref[...] = xv + proj.reshape(B, HS, WS, C)

    return pl.pallas_call(
        body,
        out_shape=jax.ShapeDtypeStruct((B, HS, WS, C), jnp.float32),
        in_specs=[
            pl.BlockSpec(memory_space=pltpu.VMEM),
            pl.BlockSpec(memory_space=pltpu.VMEM),
            pl.BlockSpec(memory_space=pltpu.VMEM),
        ],
        out_specs=pl.BlockSpec(memory_space=pltpu.VMEM),
        scratch_shapes=[
            pltpu.VMEM((2, B, C), jnp.float32),
            pltpu.VMEM((2, 2, B, C), jnp.float32),
            pltpu.VMEM((B, HS + 2, WS + 2, C), jnp.float32),
            pltpu.SemaphoreType.DMA((4,)),
            pltpu.SemaphoreType.DMA((4,)),
        ],
        compiler_params=_CompilerParams(collective_id=0),
    )(x, k, Wp)


# baseline (device time: 23939 ns/iter reference)
import jax
import jax.numpy as jnp
from jax import lax
from jax.experimental import pallas as pl
from jax.experimental.pallas import tpu as pltpu

B, HS, WS, C = 2, 64, 64, 64
GH, GW = 128, 128
EPS = 1e-5

_CompilerParams = getattr(pltpu, "CompilerParams", None) or getattr(
    pltpu, "TPUCompilerParams"
)


def kernel(x, k, Wp):
    def body(x_ref, k_ref, w_ref, out_ref, stats_s, stats_r, pad,
             send_sems, recv_sems):
        px = lax.axis_index("x")
        py = lax.axis_index("y")
        xn = (1 - px, py)
        yn = (px, 1 - py)

        bar = pltpu.get_barrier_semaphore()
        pl.semaphore_signal(bar, inc=1, device_id=xn,
                            device_id_type=pl.DeviceIdType.MESH)
        pl.semaphore_signal(bar, inc=1, device_id=yn,
                            device_id_type=pl.DeviceIdType.MESH)
        pl.semaphore_wait(bar, 2)

        xv = x_ref[...]

        stats_s[0, :, :] = jnp.sum(xv, axis=(1, 2))
        stats_s[1, :, :] = jnp.sum(xv * xv, axis=(1, 2))

        st1 = pltpu.make_async_remote_copy(
            src_ref=stats_s,
            dst_ref=stats_r.at[0],
            send_sem=send_sems.at[0],
            recv_sem=recv_sems.at[0],
            device_id=xn,
            device_id_type=pl.DeviceIdType.MESH,
        )
        st1.start()

        pad[:, 1:HS + 1, 1:WS + 1, :] = xv

        src_r = jnp.where(px == 0, HS, 1)
        dst_r = jnp.where(px == 0, 0, HS + 1)
        row_rdma = pltpu.make_async_remote_copy(
            src_ref=pad.at[:, pl.ds(src_r, 1), pl.ds(1, WS), :],
            dst_ref=pad.at[:, pl.ds(dst_r, 1), pl.ds(1, WS), :],
            send_sem=send_sems.at[2],
            recv_sem=recv_sems.at[2],
            device_id=xn,
            device_id_type=pl.DeviceIdType.MESH,
        )
        row_rdma.start()

        @pl.when(px == 0)
        def _():
            pad[:, 0:1, 1:WS + 1, :] = xv[:, 0:1, :, :]

        @pl.when(px == 1)
        def _():
            pad[:, HS + 1:HS + 2, 1:WS + 1, :] = xv[:, HS - 1:HS, :, :]

        row_rdma.wait()
        src_c = jnp.where(py == 0, WS, 1)
        dst_c = jnp.where(py == 0, 0, WS + 1)
        col_rdma = pltpu.make_async_remote_copy(
            src_ref=pad.at[:, :, pl.ds(src_c, 1), :],
            dst_ref=pad.at[:, :, pl.ds(dst_c, 1), :],
            send_sem=send_sems.at[3],
            recv_sem=recv_sems.at[3],
            device_id=yn,
            device_id_type=pl.DeviceIdType.MESH,
        )
        col_rdma.start()

        @pl.when(py == 0)
        def _():
            pad[:, :, 0:1, :] = pad[:, :, 1:2, :]

        @pl.when(py == 1)
        def _():
            pad[:, :, WS + 1:WS + 2, :] = pad[:, :, WS:WS + 1, :]

        st1.wait()
        stats_s[...] = stats_s[...] + stats_r[0]
        st2 = pltpu.make_async_remote_copy(
            src_ref=stats_s,
            dst_ref=stats_r.at[1],
            send_sem=send_sems.at[1],
            recv_sem=recv_sems.at[1],
            device_id=yn,
            device_id_type=pl.DeviceIdType.MESH,
        )
        st2.start()

        col_rdma.wait()
        hp = pad[...]
        conv = jnp.zeros((B, HS, WS, C), jnp.float32)
        for di in range(3):
            for dj in range(3):
                conv = conv + hp[:, di:di + HS, dj:dj + WS, :] * (
                    k_ref[di, dj, :].reshape(1, 1, 1, C)
                )

        st2.wait()
        total = stats_s[...] + stats_r[1]
        n = float(GH * GW)
        mean = total[0] / n
        var = total[1] / n - mean * mean
        inv = lax.rsqrt(var + EPS)
        ksum = jnp.sum(k_ref[...], axis=(0, 1))
        off = (mean * ksum.reshape(1, C)).reshape(B, 1, 1, C)
        inv_b = inv.reshape(B, 1, 1, C)

        conv = (conv - off) * inv_b
        a = conv / (1.0 + jnp.exp(-conv))
        proj = jnp.dot(
            a.reshape(B * HS * WS, C), w_ref[...],
            preferred_element_type=jnp.float32,
        )
        out_ref[...] = xv + proj.reshape(B, HS, WS, C)

    return pl.pallas_call(
        body,
        out_shape=jax.ShapeDtypeStruct((B, HS, WS, C), jnp.float32),
        in_specs=[
            pl.BlockSpec(memory_space=pltpu.VMEM),
            pl.BlockSpec(memory_space=pltpu.VMEM),
            pl.BlockSpec(memory_space=pltpu.VMEM),
        ],
        out_specs=pl.BlockSpec(memory_space=pltpu.VMEM),
        scratch_shapes=[
            pltpu.VMEM((2, B, C), jnp.float32),
            pltpu.VMEM((2, 2, B, C), jnp.float32),
            pltpu.VMEM((B, HS + 2, WS + 2, C), jnp.float32),
            pltpu.SemaphoreType.DMA((4,)),
            pltpu.SemaphoreType.DMA((4,)),
        ],
        compiler_params=_CompilerParams(collective_id=0),
    )(x, k, Wp)
